# baseline (speedup 1.0000x reference)
import jax
import jax.numpy as jnp
from jax.experimental import pallas as pl


def _copy_body(x_ref, o_ref):
    o_ref[...] = x_ref[...]


def _pallas_copy(x, block_rows):
    n, c = x.shape
    grid = (n + block_rows - 1) // block_rows
    return pl.pallas_call(
        _copy_body,
        out_shape=jax.ShapeDtypeStruct((n, c), x.dtype),
        grid=(grid,),
        in_specs=[pl.BlockSpec((block_rows, c), lambda i: (i, 0))],
        out_specs=pl.BlockSpec((block_rows, c), lambda i: (i, 0)),
    )(x)


def kernel(x_dict, edge_index, entity_emb, rel_emb):
    ne, d = entity_emb.shape
    nr, _ = rel_emb.shape
    ent2 = entity_emb.reshape(ne * d // 128, 128)
    rel2 = rel_emb.reshape(nr * d // 128, 128)
    ent_out = _pallas_copy(ent2, 4000).reshape(ne, d)
    rel_out = _pallas_copy(rel2, rel2.shape[0]).reshape(nr, d)
    return (ent_out, rel_out)
